# baseline (speedup 1.0000x reference)
import functools

import jax
import jax.numpy as jnp
from jax import lax
from jax.experimental import pallas as pl
from jax.experimental.pallas import tpu as pltpu
from jax.experimental.pallas import tpu_sc as plsc

_NC = 2
_NS = 16
_L = 16
_BE = 40
_HIGH = jax.lax.Precision.DEFAULT


def _ab_body(px_ref, wpp_ref, wwi_ref, wwj_ref, a_ref, b_ref):
    wpi = jnp.dot(wpp_ref[...], wwi_ref[...], precision=_HIGH)
    wpj = jnp.dot(wpp_ref[...], wwj_ref[...], precision=_HIGH)
    x = px_ref[...]
    a_ref[...] = jnp.dot(x, wpi, precision=_HIGH)
    b_ref[...] = jnp.dot(x, wpj, precision=_HIGH)


def _compute_ab(px, W_pp, W_wi, W_wj):
    n, d = px.shape
    blk = 2000
    w_spec = pl.BlockSpec((d, d), lambda i: (0, 0))
    row_spec = pl.BlockSpec((blk, d), lambda i: (i, 0))
    return pl.pallas_call(
        _ab_body,
        grid=(n // blk,),
        in_specs=[row_spec, w_spec, w_spec, w_spec],
        out_specs=[row_spec, row_spec],
        out_shape=[jax.ShapeDtypeStruct((n, d), jnp.float32)] * 2,
    )(px, W_pp, W_wi, W_wj)


def _sc_body(c0, nblk, a_hbm, b_hbm, diff_hbm, idxi_hbm, idxj_hbm, zeros_hbm,
             u_hbm, s2_hbm,
             ii0, ii1, ii2, ii3, ii4, ii5, ij0, ij1, ij2, ij3, ij4, ij5,
             a0, a1, a2, b0, b1, b2, d0, d1, d2, acc,
             sg0, sg1, sg2, su0, su1, su2, ss0, ss1, ss2,
             sx0, sx1, sx2, sx3, sx4, sx5):
    npad, d = zeros_hbm.shape
    e = diff_hbm.shape[0]
    cid = lax.axis_index("c")
    sid = lax.axis_index("s")
    nw = _NC * _NS
    wid = sid * _NC + cid
    rps = npad // _NS

    ii, ij = [ii0, ii1, ii2, ii3, ii4, ii5], [ij0, ij1, ij2, ij3, ij4, ij5]
    bufa, bufb, bufd = [a0, a1, a2], [b0, b1, b2], [d0, d1, d2]
    semg, semu, sems = [sg0, sg1, sg2], [su0, su1, su2], [ss0, ss1, ss2]
    semx = [sx0, sx1, sx2, sx3, sx4, sx5]
    last = nblk - 1
    base_w = wid * (e // nw) + c0 * _BE
    base_u = wid * (nblk * _BE)

    pltpu.sync_copy(zeros_hbm.at[pl.ds(sid * rps, rps)],
                    acc.at[pl.ds(sid * rps, rps)])
    plsc.subcore_barrier()

    def fire_idx(c, x):
        pltpu.async_copy(idxi_hbm.at[wid, c0 + c], ii[x], semx[x])
        pltpu.async_copy(idxj_hbm.at[wid, c0 + c], ij[x], semx[x])

    def wait_idx(c, x):
        pltpu.make_async_copy(idxi_hbm.at[wid, c0 + c], ii[x], semx[x]).wait()
        pltpu.make_async_copy(idxj_hbm.at[wid, c0 + c], ij[x], semx[x]).wait()

    def fire(c, p, x):
        pltpu.async_copy(a_hbm.at[ii[x]], bufa[p], semg[p])
        pltpu.async_copy(b_hbm.at[ij[x]], bufb[p], semg[p])
        pltpu.async_copy(diff_hbm.at[pl.ds(base_w + c * _BE, _BE)],
                         bufd[p], semg[p])

    def wait_gathers(c, p, x):
        pltpu.make_async_copy(a_hbm.at[ii[x]], bufa[p], semg[p]).wait()
        pltpu.make_async_copy(b_hbm.at[ij[x]], bufb[p], semg[p]).wait()
        pltpu.make_async_copy(diff_hbm.at[pl.ds(base_w + c * _BE, _BE)],
                              bufd[p], semg[p]).wait()

    def fire_writes(c, p, x):
        pltpu.async_copy(bufa[p], u_hbm.at[pl.ds(base_u + c * _BE, _BE)], semu[p])
        pltpu.async_copy(bufa[p], acc.at[ii[x]], sems[p], add=True)

    def wait_writes(c, p, x):
        pltpu.make_async_copy(bufa[p], u_hbm.at[pl.ds(base_u + c * _BE, _BE)],
                              semu[p]).wait()
        pltpu.make_async_copy(bufa[p], acc.at[ii[x]], sems[p]).wait()

    def compute(p):
        va, vb, vd = bufa[p], bufb[p], bufd[p]

        def row(r, car):
            for cc in range(d // _L):
                sl = pl.ds(cc * _L, _L)
                va[r, sl] = (va[r, sl] + vb[r, sl]) * vd[r, sl]
            return car
        lax.fori_loop(0, _BE, row, 0)

    fire_idx(0, 0)
    fire_idx(1, 1)
    fire_idx(2, 2)
    wait_idx(0, 0)
    fire(0, 0, 0)
    wait_idx(1, 1)
    fire(1, 1, 1)

    def outer(i, car):
        for bb in range(6):
            c = 6 * i + bb
            p = bb % 3
            x = bb
            p2, x2 = (bb + 2) % 3, (bb + 2) % 6

            @pl.when(c <= last)
            def _():
                @pl.when(c >= 1)
                def _():
                    wait_writes(c - 1, (bb + 2) % 3, (bb + 5) % 6)

                @pl.when(c + 2 <= last)
                def _():
                    wait_idx(c + 2, x2)
                    fire(c + 2, p2, x2)
                wait_gathers(c, p, x)

                @pl.when(c + 3 <= last)
                def _():
                    fire_idx(c + 3, (bb + 3) % 6)
                compute(p)
                fire_writes(c, p, x)
        return car

    lax.fori_loop(0, (nblk + 5) // 6, outer, 0)
    wait_writes(last, last % 3, last % 6)

    plsc.subcore_barrier()
    pltpu.sync_copy(acc.at[pl.ds(sid * rps, rps)],
                    s2_hbm.at[pl.ds((cid * _NS + sid) * rps, rps)])


def _sc_half(h, a, b, diff, idx_i3, idx_j3, zeros):
    e, d = diff.shape
    npad = zeros.shape[0]
    nblk = idx_i3.shape[1] // 2
    eh = e // 2
    k = pl.kernel(
        functools.partial(_sc_body, h * nblk, nblk),
        out_type=[jax.ShapeDtypeStruct((eh, d), jnp.float32),
                  jax.ShapeDtypeStruct((2 * npad, d), jnp.float32)],
        mesh=plsc.VectorSubcoreMesh(core_axis_name="c", subcore_axis_name="s"),
        scratch_types=(
            [pltpu.VMEM((_BE,), jnp.int32)] * 12
            + [pltpu.VMEM((_BE, d), jnp.float32)] * 9
            + [pltpu.VMEM_SHARED((npad, d), jnp.float32)]
            + [pltpu.SemaphoreType.DMA] * 15
        ),
    )
    return k(a, b, diff, idx_i3, idx_j3, zeros)


_IXBLK = 1000


def _ix_first_body(u_ref, wii_ref, ix_ref):
    ix_ref[...] = jnp.dot(u_ref[...], wii_ref[...], precision=_HIGH)


def _ix_first(u0, W_ii, e):
    eh, d = u0.shape
    nw = _NC * _NS
    bpt = (eh // nw) // _IXBLK
    fullbpt = (e // nw) // _IXBLK
    return pl.pallas_call(
        _ix_first_body,
        grid=(eh // _IXBLK,),
        in_specs=[pl.BlockSpec((_IXBLK, d), lambda i: (i, 0)),
                  pl.BlockSpec((d, d), lambda i: (0, 0))],
        out_specs=pl.BlockSpec(
            (_IXBLK, d), lambda i: ((i // bpt) * fullbpt + i % bpt, 0)),
        out_shape=jax.ShapeDtypeStruct((e, d), jnp.float32),
    )(u0, W_ii)


def _ix_second_body(n, u_ref, wii_ref, s20_ref, s21_ref, ixp_ref,
                    ix_ref, pxo_ref):
    wii = wii_ref[...]
    ix_ref[...] = jnp.dot(u_ref[...], wii, precision=_HIGH)

    @pl.when(pl.program_id(0) == 0)
    def _():
        npad = s20_ref.shape[0] // 2
        s = (s20_ref[:npad] + s20_ref[npad:]
             + s21_ref[:npad] + s21_ref[npad:])
        pxo_ref[...] = jnp.dot(s[:n, :], wii, precision=_HIGH)


def _ix_second(u1, W_ii, s20, s21, ix_part, n):
    eh, d = u1.shape
    e = ix_part.shape[0]
    nw = _NC * _NS
    bpt = (eh // nw) // _IXBLK
    fullbpt = (e // nw) // _IXBLK
    npad2 = s20.shape[0]
    return pl.pallas_call(
        functools.partial(_ix_second_body, n),
        grid=(eh // _IXBLK,),
        in_specs=[pl.BlockSpec((_IXBLK, d), lambda i: (i, 0)),
                  pl.BlockSpec((d, d), lambda i: (0, 0)),
                  pl.BlockSpec((npad2, d), lambda i: (0, 0)),
                  pl.BlockSpec((npad2, d), lambda i: (0, 0)),
                  pl.BlockSpec(memory_space=pl.ANY)],
        out_specs=[pl.BlockSpec(
                       (_IXBLK, d),
                       lambda i: ((i // bpt) * fullbpt + i % bpt + bpt, 0)),
                   pl.BlockSpec((n, d), lambda i: (0, 0))],
        out_shape=[jax.ShapeDtypeStruct((e, d), jnp.float32),
                   jax.ShapeDtypeStruct((n, d), jnp.float32)],
        input_output_aliases={4: 0},
    )(u1, W_ii, s20, s21, ix_part)


def kernel(px, diff, W_pp, W_wi, W_wj, W_ii, idx_i, idx_j):
    n, d = px.shape
    e = diff.shape[0]
    nw = _NC * _NS
    rps = (-(-n // _NS) + 7) // 8 * 8
    npad = rps * _NS
    nblk_t = e // (nw * _BE)
    a, b = _compute_ab(px, W_pp, W_wi, W_wj)
    zeros = jnp.zeros((npad, d), jnp.float32)
    i3 = idx_i.astype(jnp.int32).reshape(nw, nblk_t, _BE)
    j3 = idx_j.astype(jnp.int32).reshape(nw, nblk_t, _BE)
    u0, s20 = _sc_half(0, a, b, diff, i3, j3, zeros)
    u1, s21 = _sc_half(1, a, b, diff, i3, j3, zeros)
    ix_part = _ix_first(u0, W_ii, e)
    ix, px_out = _ix_second(u1, W_ii, s20, s21, ix_part, n)
    return (px_out, ix)


# speedup vs baseline: 1.1229x; 1.1229x over previous
import functools

import jax
import jax.numpy as jnp
from jax import lax
from jax.experimental import pallas as pl
from jax.experimental.pallas import tpu as pltpu
from jax.experimental.pallas import tpu_sc as plsc

_NC = 2
_NS = 16
_L = 16
_BE = 40
_HIGH = jax.lax.Precision.DEFAULT


def _ab_body(px_ref, wpp_ref, wwi_ref, wwj_ref, a_ref, b_ref):
    wpi = jnp.dot(wpp_ref[...], wwi_ref[...], precision=_HIGH)
    wpj = jnp.dot(wpp_ref[...], wwj_ref[...], precision=_HIGH)
    x = px_ref[...]
    a_ref[...] = jnp.dot(x, wpi, precision=_HIGH)
    b_ref[...] = jnp.dot(x, wpj, precision=_HIGH)


def _compute_ab(px, W_pp, W_wi, W_wj):
    n, d = px.shape
    blk = 2000
    w_spec = pl.BlockSpec((d, d), lambda i: (0, 0))
    row_spec = pl.BlockSpec((blk, d), lambda i: (i, 0))
    return pl.pallas_call(
        _ab_body,
        grid=(n // blk,),
        in_specs=[row_spec, w_spec, w_spec, w_spec],
        out_specs=[row_spec, row_spec],
        out_shape=[jax.ShapeDtypeStruct((n, d), jnp.float32)] * 2,
    )(px, W_pp, W_wi, W_wj)


def _sc_body(npad, a_hbm, b_hbm, diff_hbm, idxi_hbm, idxj_hbm,
             u_hbm, s2_hbm,
             ii0, ii1, ii2, ii3, ii4, ii5, ij0, ij1, ij2, ij3, ij4, ij5,
             a0, a1, a2, b0, b1, b2, d0, d1, d2, acc,
             sg0, sg1, sg2, su0, su1, su2, ss0, ss1, ss2,
             sx0, sx1, sx2, sx3, sx4, sx5):
    e, d = diff_hbm.shape
    cid = lax.axis_index("c")
    sid = lax.axis_index("s")
    nw = _NC * _NS
    wid = sid * _NC + cid
    rps = npad // _NS

    ii, ij = [ii0, ii1, ii2, ii3, ii4, ii5], [ij0, ij1, ij2, ij3, ij4, ij5]
    bufa, bufb, bufd = [a0, a1, a2], [b0, b1, b2], [d0, d1, d2]
    semg, semu, sems = [sg0, sg1, sg2], [su0, su1, su2], [ss0, ss1, ss2]
    semx = [sx0, sx1, sx2, sx3, sx4, sx5]
    nblk = idxi_hbm.shape[1]
    last = nblk - 1
    base_w = wid * (e // nw)

    def zrow(r, car):
        for cc in range(d // _L):
            a0[r, pl.ds(cc * _L, _L)] = jnp.zeros((_L,), jnp.float32)
        return car
    lax.fori_loop(0, _BE, zrow, 0)
    nfull = rps // _BE
    def zcopy(t, car):
        pltpu.sync_copy(a0, acc.at[pl.ds(sid * rps + t * _BE, _BE)])
        return car
    lax.fori_loop(0, nfull, zcopy, 0)
    rem = rps - nfull * _BE
    if rem:
        pltpu.sync_copy(a0.at[pl.ds(0, rem)],
                        acc.at[pl.ds(sid * rps + nfull * _BE, rem)])
    plsc.subcore_barrier()

    def fire_idx(c, x):
        pltpu.async_copy(idxi_hbm.at[wid, c], ii[x], semx[x])
        pltpu.async_copy(idxj_hbm.at[wid, c], ij[x], semx[x])

    def wait_idx(c, x):
        pltpu.make_async_copy(idxi_hbm.at[wid, c], ii[x], semx[x]).wait()
        pltpu.make_async_copy(idxj_hbm.at[wid, c], ij[x], semx[x]).wait()

    def fire(c, p, x):
        pltpu.async_copy(a_hbm.at[ii[x]], bufa[p], semg[p])
        pltpu.async_copy(b_hbm.at[ij[x]], bufb[p], semg[p])
        pltpu.async_copy(diff_hbm.at[pl.ds(base_w + c * _BE, _BE)],
                         bufd[p], semg[p])

    def wait_gathers(c, p, x):
        pltpu.make_async_copy(a_hbm.at[ii[x]], bufa[p], semg[p]).wait()
        pltpu.make_async_copy(b_hbm.at[ij[x]], bufb[p], semg[p]).wait()
        pltpu.make_async_copy(diff_hbm.at[pl.ds(base_w + c * _BE, _BE)],
                              bufd[p], semg[p]).wait()

    def fire_writes(c, p, x):
        pltpu.async_copy(bufa[p], u_hbm.at[pl.ds(base_w + c * _BE, _BE)], semu[p])
        pltpu.async_copy(bufa[p], acc.at[ii[x]], sems[p], add=True)

    def wait_writes(c, p, x):
        pltpu.make_async_copy(bufa[p], u_hbm.at[pl.ds(base_w + c * _BE, _BE)],
                              semu[p]).wait()
        pltpu.make_async_copy(bufa[p], acc.at[ii[x]], sems[p]).wait()

    def compute(p):
        va, vb, vd = bufa[p], bufb[p], bufd[p]

        def row(r, car):
            for cc in range(d // _L):
                sl = pl.ds(cc * _L, _L)
                va[r, sl] = (va[r, sl] + vb[r, sl]) * vd[r, sl]
            return car
        lax.fori_loop(0, _BE, row, 0)

    fire_idx(0, 0)
    fire_idx(1, 1)
    fire_idx(2, 2)
    wait_idx(0, 0)
    fire(0, 0, 0)
    wait_idx(1, 1)
    fire(1, 1, 1)

    def outer(i, car):
        for bb in range(6):
            c = 6 * i + bb
            p = bb % 3
            x = bb
            p2, x2 = (bb + 2) % 3, (bb + 2) % 6

            @pl.when(c <= last)
            def _():
                @pl.when(c >= 1)
                def _():
                    wait_writes(c - 1, (bb + 2) % 3, (bb + 5) % 6)

                @pl.when(c + 2 <= last)
                def _():
                    wait_idx(c + 2, x2)
                    fire(c + 2, p2, x2)
                wait_gathers(c, p, x)

                @pl.when(c + 3 <= last)
                def _():
                    fire_idx(c + 3, (bb + 3) % 6)
                compute(p)
                fire_writes(c, p, x)
        return car

    lax.fori_loop(0, (nblk + 5) // 6, outer, 0)
    wait_writes(last, last % 3, last % 6)

    plsc.subcore_barrier()
    pltpu.sync_copy(acc.at[pl.ds(sid * rps, rps)],
                    s2_hbm.at[pl.ds((cid * _NS + sid) * rps, rps)])


def _sc_gather_scale_scatter(npad, a, b, diff, idx_i3, idx_j3):
    e, d = diff.shape
    k = pl.kernel(
        functools.partial(_sc_body, npad),
        out_type=[jax.ShapeDtypeStruct((e, d), jnp.float32),
                  jax.ShapeDtypeStruct((2 * npad, d), jnp.float32)],
        mesh=plsc.VectorSubcoreMesh(core_axis_name="c", subcore_axis_name="s"),
        scratch_types=(
            [pltpu.VMEM((_BE,), jnp.int32)] * 12
            + [pltpu.VMEM((_BE, d), jnp.float32)] * 9
            + [pltpu.VMEM_SHARED((npad, d), jnp.float32)]
            + [pltpu.SemaphoreType.DMA] * 15
        ),
    )
    return k(a, b, diff, idx_i3, idx_j3)


def _ix_body(n, u_ref, wii_ref, s2_ref, ix_ref, pxo_ref):
    wii = wii_ref[...]
    ix_ref[...] = jnp.dot(u_ref[...], wii, precision=_HIGH)

    @pl.when(pl.program_id(0) == 0)
    def _():
        npad = s2_ref.shape[0] // 2
        s = s2_ref[:npad] + s2_ref[npad:]
        pxo_ref[...] = jnp.dot(s[:n, :], wii, precision=_HIGH)


def _compute_ix_out(u, W_ii, s2, n):
    e, d = u.shape
    npad2 = s2.shape[0]
    blk = 2560
    return pl.pallas_call(
        functools.partial(_ix_body, n),
        grid=(e // blk,),
        in_specs=[pl.BlockSpec((blk, d), lambda i: (i, 0)),
                  pl.BlockSpec((d, d), lambda i: (0, 0)),
                  pl.BlockSpec((npad2, d), lambda i: (0, 0))],
        out_specs=[pl.BlockSpec((blk, d), lambda i: (i, 0)),
                   pl.BlockSpec((n, d), lambda i: (0, 0))],
        out_shape=[jax.ShapeDtypeStruct((e, d), jnp.float32),
                   jax.ShapeDtypeStruct((n, d), jnp.float32)],
    )(u, W_ii, s2)


def kernel(px, diff, W_pp, W_wi, W_wj, W_ii, idx_i, idx_j):
    n, d = px.shape
    e = diff.shape[0]
    nw = _NC * _NS
    rps = (-(-n // _NS) + 7) // 8 * 8
    npad = rps * _NS
    nblk_t = e // (nw * _BE)
    a, b = _compute_ab(px, W_pp, W_wi, W_wj)
    i3 = idx_i.astype(jnp.int32).reshape(nw, nblk_t, _BE)
    j3 = idx_j.astype(jnp.int32).reshape(nw, nblk_t, _BE)
    u, s2 = _sc_gather_scale_scatter(npad, a, b, diff, i3, j3)
    ix, px_out = _compute_ix_out(u, W_ii, s2, n)
    return (px_out, ix)
